# Initial kernel scaffold; baseline (speedup 1.0000x reference)
import jax
import jax.numpy as jnp
from jax.experimental import pallas as pl


def kernel(X_batch, rows, cols, vals, coeffs, t_mid, t_half):
    raise NotImplementedError("write your pallas kernel here")


# speedup vs baseline: 5.2032x; 5.2032x over previous
import functools

import jax
import jax.numpy as jnp
from jax import lax
from jax.experimental import pallas as pl
from jax.experimental.pallas import tpu as pltpu
from jax.experimental.pallas import tpu_sc as plsc

N = 16384
F = 64
E = 268435
DEGREE = 20
NC = 2
NS = 16
NW = NC * NS
CH = 128
NCH = 68
EPW = NCH * CH
E_PAD = NW * EPW
NBUF = 4
RPT = N // NS

_sc_mesh = plsc.VectorSubcoreMesh(core_axis_name="c", subcore_axis_name="s")


@functools.partial(
    pl.kernel,
    out_type=jax.ShapeDtypeStruct((NC, N, F), jnp.float32),
    mesh=_sc_mesh,
    scratch_types=[
        pltpu.VMEM_SHARED((N, F), jnp.float32),
        pltpu.VMEM((NCH, CH), jnp.int32),
        pltpu.VMEM((NCH, CH), jnp.int32),
        pltpu.VMEM((NCH, CH), jnp.float32),
        pltpu.VMEM((NBUF, CH, F), jnp.float32),
    ]
    + [pltpu.SemaphoreType.DMA] * (2 * NBUF),
    compiler_params=pltpu.CompilerParams(use_tc_tiling_on_sc=False),
)
def _spmm_sc(table, gidx, sidx, valx, parts,
             acc, gidx_t, sidx_t, val_t, gb, *sems):
    gsems = sems[:NBUF]
    ssems = sems[NBUF:]
    cid = lax.axis_index("c")
    sid = lax.axis_index("s")
    wid = cid * NS + sid

    pltpu.sync_copy(gidx.at[wid], gidx_t)
    pltpu.sync_copy(sidx.at[wid], sidx_t)
    pltpu.sync_copy(valx.at[wid], val_t)

    z16 = jnp.zeros((16,), jnp.float32)

    @pl.loop(0, CH)
    def _(r):
        for j in range(4):
            gb[0, r, pl.ds(j * 16, 16)] = z16

    for h in range(RPT // CH):
        pltpu.sync_copy(gb.at[0], acc.at[pl.ds(sid * RPT + h * CH, CH)])

    def start_gather(c, k):
        pltpu.async_copy(table.at[gidx_t.at[c]], gb.at[k], gsems[k])

    def wait_gather(k):
        pltpu.make_async_copy(table.at[gidx_t.at[0]], gb.at[k], gsems[k]).wait()

    def start_scatter(c, k):
        pltpu.async_copy(gb.at[k], acc.at[sidx_t.at[c]], ssems[k], add=True)

    def wait_scatter(k):
        pltpu.make_async_copy(gb.at[k], acc.at[sidx_t.at[0]], ssems[k]).wait()

    def scale(c, k):
        @pl.loop(0, CH // 16)
        def _(q):
            vv = val_t[c, pl.ds(q * 16, 16)]
            for j in range(16):
                v = vv[j]
                e = q * 16 + j
                for jj in range(4):
                    sl = pl.ds(jj * 16, 16)
                    gb[k, e, sl] = gb[k, e, sl] * v

    plsc.subcore_barrier()

    start_gather(0, 0)
    start_gather(1, 1)

    @pl.loop(0, NCH // NBUF)
    def _(t):
        for p in range(NBUF):
            k = p
            c = t * NBUF + p
            wait_gather(k)
            scale(c, k)
            start_scatter(c, k)
            nb = c + (NBUF - 2)
            kb = (p + NBUF - 2) % NBUF

            @pl.when(nb < NCH)
            def _():
                @pl.when(nb >= NBUF)
                def _():
                    wait_scatter(kb)

                start_gather(nb, kb)

    for k in range(NBUF):
        wait_scatter(k)

    plsc.subcore_barrier()

    for h in range(2):
        sl = pl.ds(sid * RPT + h * 512, 512)
        pltpu.sync_copy(acc.at[sl], parts.at[cid, sl])


def _cheby_body(par, p0, p1, tc, tp, oi, tn, on):
    a = par[0]
    b = par[1]
    cc = par[2]
    ck = par[3]
    dd = par[4]
    w = p0[...] + p1[...]
    t = a * w + b * tc[...] + cc * tp[...]
    tn[...] = t
    on[...] = dd * oi[...] + ck * t


_cheby = pl.pallas_call(
    _cheby_body,
    grid=(N // RPT,),
    in_specs=[pl.BlockSpec(memory_space=pltpu.SMEM)]
    + [pl.BlockSpec((RPT, F), lambda i: (i, 0))] * 5,
    out_specs=[pl.BlockSpec((RPT, F), lambda i: (i, 0))] * 2,
    out_shape=[jax.ShapeDtypeStruct((N, F), jnp.float32)] * 2,
)


def _add2_body(p0, p1, o):
    o[...] = p0[...] + p1[...]


_add2 = pl.pallas_call(
    _add2_body,
    grid=(N // RPT,),
    in_specs=[pl.BlockSpec((RPT, F), lambda i: (i, 0))] * 2,
    out_specs=pl.BlockSpec((RPT, F), lambda i: (i, 0)),
    out_shape=jax.ShapeDtypeStruct((N, F), jnp.float32),
)


def _pack(g, s, v):
    gp = jnp.zeros((E_PAD,), jnp.int32).at[:E].set(g).reshape(NW, NCH, CH)
    sp = jnp.zeros((E_PAD,), jnp.int32).at[:E].set(s).reshape(NW, NCH, CH)
    vp = jnp.zeros((E_PAD,), jnp.float32).at[:E].set(v).reshape(NW, NCH, CH)
    return gp, sp, vp


def kernel(X_batch, rows, cols, vals, coeffs, t_mid, t_half):
    rows = rows.astype(jnp.int32)
    cols = cols.astype(jnp.int32)
    vals = vals.astype(jnp.float32)
    gA, sA, vA = _pack(cols, rows, vals)
    gB, sB, vB = _pack(rows, cols, vals)

    Xt = X_batch.T
    inv_h = (1.0 / t_half).astype(jnp.float32)
    m = t_mid.astype(jnp.float32)
    zero = jnp.float32(0.0)
    one = jnp.float32(1.0)

    Tprev, Tcur, out = Xt, Xt, Xt
    for k in range(1, DEGREE + 1):
        parts1 = _spmm_sc(Tcur, gA, sA, vA)
        u = _add2(parts1[0], parts1[1])
        parts2 = _spmm_sc(u, gB, sB, vB)
        if k == 1:
            par = jnp.stack([inv_h, -m * inv_h, zero, coeffs[1], coeffs[0],
                             zero, zero, zero])
        else:
            par = jnp.stack([2.0 * inv_h, -2.0 * m * inv_h, -one, coeffs[k],
                             one, zero, zero, zero])
        tn, out = _cheby(par.astype(jnp.float32), parts2[0], parts2[1],
                         Tcur, Tprev, out)
        Tprev, Tcur = Tcur, tn
    return out.T
